# baseline (speedup 1.0000x reference)
import functools

import jax
import jax.numpy as jnp
from jax import lax
from jax.experimental import pallas as pl
from jax.experimental.pallas import tpu as pltpu
from jax.experimental.pallas import tpu_sc as plsc

VOCAB = 1000000
N = 819200
DIM = 32
NUM_CORES = 2
NUM_SUBCORES = 16
NW = NUM_CORES * NUM_SUBCORES

KCOL = 4
SLABW = 128 * KCOL
NSLAB = VOCAB // SLABW
SLAB_PER_W = (VOCAB // SLABW) // NW
NFULL = VOCAB // 128
TAIL0 = NFULL * 128
NTAIL = VOCAB - TAIL0

B_PER_W = N // NW
CHUNK = 640
NCHUNK = B_PER_W // CHUNK
SUPER = 2
NSUPER = NCHUNK // SUPER
NGRP = N // 128
CGRP = CHUNK // 128
SGRP = CGRP * SUPER

_MESH = dict(core_axis_name="c", subcore_axis_name="s")


def _detile_table(tt, tail):
    mesh = plsc.VectorSubcoreMesh(**_MESH)

    scratch = [
        pltpu.VMEM((KCOL, DIM, 128), jnp.float32),
        pltpu.VMEM((KCOL, DIM, 128), jnp.float32),
        pltpu.VMEM((SLABW * DIM,), jnp.float32),
        pltpu.VMEM((SLABW * DIM,), jnp.float32),
        pltpu.VMEM((NTAIL * DIM,), jnp.float32),
        pltpu.SemaphoreType.DMA,
        pltpu.SemaphoreType.DMA,
        pltpu.SemaphoreType.DMA,
        pltpu.SemaphoreType.DMA,
    ]

    @functools.partial(
        pl.kernel,
        mesh=mesh,
        out_type=jax.ShapeDtypeStruct((VOCAB * DIM,), jnp.float32),
        scratch_types=scratch,
        compiler_params=pltpu.CompilerParams(needs_layout_passes=False),
    )
    def k(tt_hbm, tail_hbm, tl_hbm, s0, s1, n0, n1, tv, rs0, rs1, ws0, ws1):
        wid = lax.axis_index("s") * NUM_CORES + lax.axis_index("c")
        slab0 = wid * SLAB_PER_W
        lane = lax.iota(jnp.int32, 16)
        slabs, nats = (s0, s1), (n0, n1)
        rsems, wsems = (rs0, rs1), (ws0, ws1)

        def rd(sl, b):
            cps = []
            for ctl in range(KCOL):
                o = pl.multiple_of((sl * KCOL + ctl) * 128, 128)
                cps.append(pltpu.make_async_copy(
                    tt_hbm.at[:, pl.ds(o, 128)], slabs[b].at[ctl], rsems[b]))
            return cps

        def wr(sl, b):
            return pltpu.make_async_copy(
                nats[b], tl_hbm.at[pl.ds(sl * (SLABW * DIM), SLABW * DIM)],
                wsems[b])

        def transpose(slab, nat):
            @plsc.parallel_loop(0, SLABW // 16, unroll=4)
            def tr(x0):
                x16 = x0 * 16 + lane
                ctl16 = lax.shift_right_logical(x16, 7)
                ii16 = x16 & 127
                base = x16 * DIM
                for c0 in range(DIM):
                    c16 = (c0 + lane) & (DIM - 1)
                    val = plsc.load_gather(slab, [ctl16, c16, ii16])
                    plsc.store_scatter(nat, [base + c16], val)

        for cp in rd(slab0, 0) + rd(slab0 + 1, 1):
            cp.start()

        def pair(p, _):
            for h in range(2):
                i = 2 * p + h
                sl = slab0 + i
                for cp in rd(sl, h):
                    cp.wait()
                pl.when(p > 0)(lambda: wr(sl - 2, h).wait())
                transpose(slabs[h], nats[h])

                def fire_read():
                    for cp in rd(sl + 2, h):
                        cp.start()
                pl.when(p < SLAB_PER_W // 2 - 1)(fire_read)
                wr(sl, h).start()
            return ()

        npair = SLAB_PER_W // 2
        lax.fori_loop(0, npair, pair, ())
        wr(slab0 + 2 * npair - 2, 0).wait()
        wr(slab0 + 2 * npair - 1, 1).wait()

        def process_sync(sl):
            for cp in rd(sl, 0):
                cp.start()
            for cp in rd(sl, 0):
                cp.wait()
            transpose(slabs[0], nats[0])
            wr(sl, 0).start()
            wr(sl, 0).wait()

        if SLAB_PER_W % 2:
            process_sync(slab0 + SLAB_PER_W - 1)
        pl.when(wid == 0)(lambda: process_sync(NW * SLAB_PER_W))

        def tail_copy():
            pltpu.sync_copy(tail_hbm, tv)
            pltpu.sync_copy(
                tv, tl_hbm.at[pl.ds(TAIL0 * DIM, NTAIL * DIM)])
        pl.when(wid == NW - 1)(tail_copy)

    return k(tt, tail)


def _sc_gather(inputs, table_lin):
    mesh = plsc.VectorSubcoreMesh(**_MESH)

    scratch = [
        pltpu.VMEM((CHUNK,), jnp.int32),
        pltpu.VMEM((CHUNK,), jnp.int32),
        pltpu.VMEM((CHUNK, DIM), jnp.float32),
        pltpu.VMEM((CHUNK, DIM), jnp.float32),
        pltpu.VMEM((DIM, SGRP, 128), jnp.float32),
        pltpu.SemaphoreType.DMA,
        pltpu.SemaphoreType.DMA,
        pltpu.SemaphoreType.DMA,
    ]

    @functools.partial(
        pl.kernel,
        mesh=mesh,
        out_type=jax.ShapeDtypeStruct((4, NGRP, 8, 128), jnp.float32),
        scratch_types=scratch,
        compiler_params=pltpu.CompilerParams(
            use_tc_tiling_on_sc=False, needs_layout_passes=False),
    )
    def k(idx_hbm, t_hbm, out_hbm, i0, i1, r0, r1, nat, gs0, gs1, ws):
        wid = lax.axis_index("s") * NUM_CORES + lax.axis_index("c")
        base = wid * B_PER_W
        lane = lax.iota(jnp.int32, 16)
        idxs, rows = (i0, i1), (r0, r1)
        gsems = (gs0, gs1)

        def gather(b):
            return pltpu.make_async_copy(t_hbm.at[idxs[b]], rows[b], gsems[b])

        def writes(sp):
            g0 = wid * (B_PER_W // 128) + sp * SGRP
            return [
                pltpu.make_async_copy(
                    nat.at[c],
                    out_hbm.at[c // 8, pl.ds(g0, SGRP), c % 8, :], ws)
                for c in range(DIM)
            ]

        def transpose(rv, q):
            @plsc.parallel_loop(0, CHUNK // 16, unroll=4)
            def tr(s):
                j16 = s * 16 + lane
                g16 = lax.shift_right_logical(j16, 7) + q * CGRP
                i16 = j16 & 127
                for c0 in range(DIM):
                    c16 = (c0 + lane) & (DIM - 1)
                    val = plsc.load_gather(rv, [j16, c16])
                    plsc.store_scatter(nat, [c16, g16, i16], val)

        pltpu.sync_copy(idx_hbm.at[pl.ds(base, CHUNK)], i0)
        gather(0).start()

        def super_body(sp, _):
            for q in range(SUPER):
                i = sp * SUPER + q
                b = q & 1
                gather(b).wait()

                def fire_next():
                    off = base + (i + 1) * CHUNK
                    pltpu.sync_copy(idx_hbm.at[pl.ds(off, CHUNK)],
                                    idxs[1 - b])
                    gather(1 - b).start()
                if q < SUPER - 1:
                    fire_next()
                else:
                    pl.when(sp < NSUPER - 1)(fire_next)

                if q == 0:
                    def drain():
                        for cp in writes(sp - 1):
                            cp.wait()
                    pl.when(sp > 0)(drain)

                transpose(rows[b], q)
            for cp in writes(sp):
                cp.start()
            return ()

        lax.fori_loop(0, NSUPER, super_body, ())
        for cp in writes(NSUPER - 1):
            cp.wait()

    return k(inputs, table_lin)


def kernel(inputs, table):
    tt = table.T
    tail = lax.slice(table, (TAIL0, 0), (VOCAB, DIM)).reshape(NTAIL * DIM)
    t_lin = _detile_table(tt, tail)
    out4 = _sc_gather(inputs, t_lin.reshape(VOCAB, DIM))
    return out4.transpose(1, 3, 0, 2).reshape(N, DIM)


# speedup vs baseline: 1.6013x; 1.6013x over previous
import functools

import jax
import jax.numpy as jnp
from jax import lax
from jax.experimental import pallas as pl
from jax.experimental.pallas import tpu as pltpu
from jax.experimental.pallas import tpu_sc as plsc

VOCAB = 1000000
N = 819200
DIM = 32
NUM_CORES = 2
NUM_SUBCORES = 16
NW = NUM_CORES * NUM_SUBCORES

KCOL = 4
SLABW = 128 * KCOL
NSLAB = VOCAB // SLABW
SLAB_PER_W = (VOCAB // SLABW) // NW
NFULL = VOCAB // 128
TAIL0 = NFULL * 128
NTAIL = VOCAB - TAIL0

B_PER_W = N // NW
CHUNK = 640
NCHUNK = B_PER_W // CHUNK
SUPER = 2
NSUPER = NCHUNK // SUPER
NGRP = N // 128
CGRP = CHUNK // 128
SGRP = CGRP * SUPER

_MESH = dict(core_axis_name="c", subcore_axis_name="s")


def _detile_table(tt, tail):
    mesh = plsc.VectorSubcoreMesh(**_MESH)

    scratch = [
        pltpu.VMEM((KCOL, DIM, 128), jnp.float32),
        pltpu.VMEM((KCOL, DIM, 128), jnp.float32),
        pltpu.VMEM((SLABW * DIM,), jnp.float32),
        pltpu.VMEM((SLABW * DIM,), jnp.float32),
        pltpu.VMEM((NTAIL * DIM,), jnp.float32),
        pltpu.SemaphoreType.DMA,
        pltpu.SemaphoreType.DMA,
        pltpu.SemaphoreType.DMA,
        pltpu.SemaphoreType.DMA,
    ]

    @functools.partial(
        pl.kernel,
        mesh=mesh,
        out_type=jax.ShapeDtypeStruct((VOCAB * DIM,), jnp.float32),
        scratch_types=scratch,
        compiler_params=pltpu.CompilerParams(needs_layout_passes=False),
    )
    def k(tt_hbm, tail_hbm, tl_hbm, s0, s1, n0, n1, tv, rs0, rs1, ws0, ws1):
        wid = lax.axis_index("s") * NUM_CORES + lax.axis_index("c")
        slab0 = wid * SLAB_PER_W
        lane = lax.iota(jnp.int32, 16)
        slabs, nats = (s0, s1), (n0, n1)
        rsems, wsems = (rs0, rs1), (ws0, ws1)

        def rd(sl, b):
            cps = []
            for ctl in range(KCOL):
                o = pl.multiple_of((sl * KCOL + ctl) * 128, 128)
                cps.append(pltpu.make_async_copy(
                    tt_hbm.at[:, pl.ds(o, 128)], slabs[b].at[ctl], rsems[b]))
            return cps

        def wr(sl, b):
            return pltpu.make_async_copy(
                nats[b], tl_hbm.at[pl.ds(sl * (SLABW * DIM), SLABW * DIM)],
                wsems[b])

        def transpose(slab, nat):
            @plsc.parallel_loop(0, SLABW // 16, unroll=2)
            def tr(x0):
                x16 = x0 * 16 + lane
                ctl16 = lax.shift_right_logical(x16, 7)
                ii16 = x16 & 127
                base = x16 * DIM
                for c0 in range(DIM):
                    c16 = (c0 + lane) & (DIM - 1)
                    val = plsc.load_gather(slab, [ctl16, c16, ii16])
                    plsc.store_scatter(nat, [base + c16], val)

        for cp in rd(slab0, 0) + rd(slab0 + 1, 1):
            cp.start()

        def pair(p, _):
            for h in range(2):
                i = 2 * p + h
                sl = slab0 + i
                for cp in rd(sl, h):
                    cp.wait()
                pl.when(p > 0)(lambda: wr(sl - 2, h).wait())
                transpose(slabs[h], nats[h])

                def fire_read():
                    for cp in rd(sl + 2, h):
                        cp.start()
                pl.when(p < SLAB_PER_W // 2 - 1)(fire_read)
                wr(sl, h).start()
            return ()

        npair = SLAB_PER_W // 2
        lax.fori_loop(0, npair, pair, ())
        wr(slab0 + 2 * npair - 2, 0).wait()
        wr(slab0 + 2 * npair - 1, 1).wait()

        def process_sync(sl):
            for cp in rd(sl, 0):
                cp.start()
            for cp in rd(sl, 0):
                cp.wait()
            transpose(slabs[0], nats[0])
            wr(sl, 0).start()
            wr(sl, 0).wait()

        if SLAB_PER_W % 2:
            process_sync(slab0 + SLAB_PER_W - 1)
        pl.when(wid == 0)(lambda: process_sync(NW * SLAB_PER_W))

        def tail_copy():
            pltpu.sync_copy(tail_hbm, tv)
            pltpu.sync_copy(
                tv, tl_hbm.at[pl.ds(TAIL0 * DIM, NTAIL * DIM)])
        pl.when(wid == NW - 1)(tail_copy)

    return k(tt, tail)


def _sc_gather(inputs, table_lin):
    mesh = plsc.VectorSubcoreMesh(**_MESH)

    scratch = [
        pltpu.VMEM((CHUNK,), jnp.int32),
        pltpu.VMEM((CHUNK,), jnp.int32),
        pltpu.VMEM((CHUNK, DIM), jnp.float32),
        pltpu.VMEM((CHUNK, DIM), jnp.float32),
        pltpu.VMEM((DIM, SGRP, 128), jnp.float32),
        pltpu.SemaphoreType.DMA,
        pltpu.SemaphoreType.DMA,
        pltpu.SemaphoreType.DMA,
    ]

    @functools.partial(
        pl.kernel,
        mesh=mesh,
        out_type=jax.ShapeDtypeStruct((4, NGRP, 8, 128), jnp.float32),
        scratch_types=scratch,
        compiler_params=pltpu.CompilerParams(
            use_tc_tiling_on_sc=False, needs_layout_passes=False),
    )
    def k(idx_hbm, t_hbm, out_hbm, i0, i1, r0, r1, nat, gs0, gs1, ws):
        wid = lax.axis_index("s") * NUM_CORES + lax.axis_index("c")
        base = wid * B_PER_W
        lane = lax.iota(jnp.int32, 16)
        idxs, rows = (i0, i1), (r0, r1)
        gsems = (gs0, gs1)

        def gather(b):
            return pltpu.make_async_copy(t_hbm.at[idxs[b]], rows[b], gsems[b])

        def writes(sp):
            g0 = wid * (B_PER_W // 128) + sp * SGRP
            return [
                pltpu.make_async_copy(
                    nat.at[c],
                    out_hbm.at[c // 8, pl.ds(g0, SGRP), c % 8, :], ws)
                for c in range(DIM)
            ]

        def transpose(rv, q):
            @plsc.parallel_loop(0, CHUNK // 16, unroll=2)
            def tr(s):
                j16 = s * 16 + lane
                g16 = lax.shift_right_logical(j16, 7) + q * CGRP
                i16 = j16 & 127
                for c0 in range(DIM):
                    c16 = (c0 + lane) & (DIM - 1)
                    val = plsc.load_gather(rv, [j16, c16])
                    plsc.store_scatter(nat, [c16, g16, i16], val)

        pltpu.sync_copy(idx_hbm.at[pl.ds(base, CHUNK)], i0)
        gather(0).start()

        def super_body(sp, _):
            for q in range(SUPER):
                i = sp * SUPER + q
                b = q & 1
                gather(b).wait()

                def fire_next():
                    off = base + (i + 1) * CHUNK
                    pltpu.sync_copy(idx_hbm.at[pl.ds(off, CHUNK)],
                                    idxs[1 - b])
                    gather(1 - b).start()
                if q < SUPER - 1:
                    fire_next()
                else:
                    pl.when(sp < NSUPER - 1)(fire_next)

                if q == 0:
                    def drain():
                        for cp in writes(sp - 1):
                            cp.wait()
                    pl.when(sp > 0)(drain)

                transpose(rows[b], q)
            for cp in writes(sp):
                cp.start()
            return ()

        lax.fori_loop(0, NSUPER, super_body, ())
        for cp in writes(NSUPER - 1):
            cp.wait()

    return k(inputs, table_lin)


def kernel(inputs, table):
    tt = table.T
    tail = lax.slice(table, (TAIL0, 0), (VOCAB, DIM)).reshape(NTAIL * DIM)
    t_lin = _detile_table(tt, tail)
    out4 = _sc_gather(inputs, t_lin.reshape(VOCAB, DIM))
    return out4.transpose(1, 3, 0, 2).reshape(N, DIM)
